# baseline (device time: 16536 ns/iter reference)
import jax
import jax.numpy as jnp
from jax import lax
from jax.experimental import pallas as pl
from jax.experimental.pallas import tpu as pltpu

TAU = 24

_SCHED = [
    (0, 40, "y"),
    (128, 40, "z"),
    (40, 40, "y"),
    (168, 40, "z"),
    (80, 24, "y"),
    (208, 24, "z"),
    (104, TAU, "own"),
    (232, TAU, "own"),
    (104, TAU, "extra"),
    (232, TAU, "extra"),
]
_N_FWD = sum(1 for _, _, r in _SCHED if r in ("y", "z"))
_FWD_HALF = 128 - TAU


def kernel(partial, gamma):
    xs, m2, d = partial.shape
    m = m2 // 2
    q = m // 2

    n_comm = sum(n for _, n, _ in _SCHED)
    g2 = gamma.reshape(1, d)

    def body(x_ref, g_ref, out_ref, xv_send, xv_loc, gv, ov, comm_ref,
             in_sems, out_sems, p1_send, p1_recv, p2_send, p2_recv):
        my_x = lax.axis_index("x")
        my_y = lax.axis_index("y")
        my_z = lax.axis_index("z")
        s = (my_y + my_z) % 2

        x_part = (1 - my_x, my_y, my_z)
        y_part = (my_x, 1 - my_y, my_z)
        z_part = (my_x, my_y, 1 - my_z)

        cp_send = pltpu.make_async_copy(
            x_ref.at[0, pl.ds((1 - my_x) * m, m), :], xv_send, in_sems.at[0])
        cp_loc = pltpu.make_async_copy(
            x_ref.at[0, pl.ds(my_x * m, m), :], xv_loc, in_sems.at[1])
        cp_g = pltpu.make_async_copy(g_ref, gv, in_sems.at[2])
        cp_send.start()
        cp_loc.start()
        cp_g.start()

        def half_off(off, role):
            base = (1 - s) * q if role == "extra" else s * q
            return base + off

        barrier = pltpu.get_barrier_semaphore()
        for dev in (x_part, y_part, z_part):
            pl.semaphore_signal(
                barrier, inc=1, device_id=dev,
                device_id_type=pl.DeviceIdType.MESH,
            )
        pl.semaphore_wait(barrier, 3)

        cp_send.wait()
        p1 = []
        coff = 0
        for c, (off, n, role) in enumerate(_SCHED):
            ho = half_off(off, role)
            r = pltpu.make_async_remote_copy(
                src_ref=xv_send.at[pl.ds(ho, n), :],
                dst_ref=comm_ref.at[pl.ds(coff, n), :],
                send_sem=p1_send.at[c],
                recv_sem=p1_recv.at[c],
                device_id=x_part,
                device_id_type=pl.DeviceIdType.MESH,
            )
            r.start()
            p1.append((r, coff))
            coff += n

        cp_loc.wait()
        cp_g.wait()

        p2 = []
        for c, (off, n, role) in enumerate(_SCHED):
            rdma, coff = p1[c]
            rdma.wait_recv()
            ho = half_off(off, role)
            red = xv_loc[pl.ds(ho, n), :] + comm_ref[pl.ds(coff, n), :]
            rinv = lax.rsqrt(jnp.mean(red * red, axis=-1, keepdims=True) + 1e-6)
            ov[pl.ds(ho, n), :] = red * rinv * gv[...]
            if role in ("y", "z"):
                fc = len(p2)
                r = pltpu.make_async_remote_copy(
                    src_ref=ov.at[pl.ds(ho, n), :],
                    dst_ref=ov.at[pl.ds(ho, n), :],
                    send_sem=p2_send.at[fc],
                    recv_sem=p2_recv.at[fc],
                    device_id=y_part if role == "y" else z_part,
                    device_id_type=pl.DeviceIdType.MESH,
                )
                r.start()
                p2.append(r)

        o1 = pltpu.make_async_copy(
            ov.at[pl.ds(s * q, q), :], out_ref.at[pl.ds(s * q, q), :],
            out_sems.at[0])
        o2a = pltpu.make_async_copy(
            ov.at[pl.ds((1 - s) * q + _FWD_HALF, TAU), :],
            out_ref.at[pl.ds((1 - s) * q + _FWD_HALF, TAU), :],
            out_sems.at[1])
        o2b = pltpu.make_async_copy(
            ov.at[pl.ds((1 - s) * q + q // 2 + _FWD_HALF, TAU), :],
            out_ref.at[pl.ds((1 - s) * q + q // 2 + _FWD_HALF, TAU), :],
            out_sems.at[4])
        o1.start()
        o2a.start()
        o2b.start()

        for r in p2[:-1]:
            r.wait_recv()
        o3 = pltpu.make_async_copy(
            ov.at[pl.ds((1 - s) * q, _FWD_HALF), :],
            out_ref.at[pl.ds((1 - s) * q, _FWD_HALF), :],
            out_sems.at[2])
        o3.start()
        p2[-1].wait_recv()
        o4 = pltpu.make_async_copy(
            ov.at[pl.ds((1 - s) * q + q // 2, _FWD_HALF), :],
            out_ref.at[pl.ds((1 - s) * q + q // 2, _FWD_HALF), :],
            out_sems.at[3])
        o4.start()

        o1.wait()
        o2a.wait()
        o2b.wait()
        o3.wait()
        o4.wait()
        for r, _ in p1:
            r.wait_send()
        for r in p2:
            r.wait_send()

    return pl.pallas_call(
        body,
        out_shape=jax.ShapeDtypeStruct((m, d), jnp.float32),
        in_specs=[
            pl.BlockSpec(memory_space=pl.ANY),
            pl.BlockSpec(memory_space=pl.ANY),
        ],
        out_specs=pl.BlockSpec(memory_space=pl.ANY),
        scratch_shapes=[
            pltpu.VMEM((m, d), jnp.float32),
            pltpu.VMEM((m, d), jnp.float32),
            pltpu.VMEM((1, d), jnp.float32),
            pltpu.VMEM((m, d), jnp.float32),
            pltpu.VMEM((n_comm, d), jnp.float32),
            pltpu.SemaphoreType.DMA((3,)),
            pltpu.SemaphoreType.DMA((5,)),
            pltpu.SemaphoreType.DMA((len(_SCHED),)),
            pltpu.SemaphoreType.DMA((len(_SCHED),)),
            pltpu.SemaphoreType.DMA((_N_FWD,)),
            pltpu.SemaphoreType.DMA((_N_FWD,)),
        ],
        compiler_params=pltpu.CompilerParams(collective_id=0),
    )(partial, g2)


# device time: 15480 ns/iter; 1.0682x vs baseline; 1.0682x over previous
import jax
import jax.numpy as jnp
from jax import lax
from jax.experimental import pallas as pl
from jax.experimental.pallas import tpu as pltpu

TAU = 24

_SCHED = [
    (0, 48, "y"),
    (128, 48, "z"),
    (48, 40, "y"),
    (176, 40, "z"),
    (88, 16, "y"),
    (216, 16, "z"),
    (104, TAU, "own"),
    (232, TAU, "own"),
    (104, TAU, "extra"),
    (232, TAU, "extra"),
]
_N_FWD = sum(1 for _, _, r in _SCHED if r in ("y", "z"))
_FWD_HALF = 128 - TAU


def kernel(partial, gamma):
    xs, m2, d = partial.shape
    m = m2 // 2
    q = m // 2

    n_comm = sum(n for _, n, _ in _SCHED)

    def body(x_ref, g_ref, out_ref, xv_send, xv_loc, gv, ov, comm_ref,
             in_sems, out_sems, yz_sem, p1_send, p1_recv, p2_send, p2_recv):
        my_x = lax.axis_index("x")
        my_y = lax.axis_index("y")
        my_z = lax.axis_index("z")
        s = (my_y + my_z) % 2

        x_part = (1 - my_x, my_y, my_z)
        y_part = (my_x, 1 - my_y, my_z)
        z_part = (my_x, my_y, 1 - my_z)

        cp_send = pltpu.make_async_copy(
            x_ref.at[0, pl.ds((1 - my_x) * m, m), :], xv_send, in_sems.at[0])
        cp_loc = pltpu.make_async_copy(
            x_ref.at[0, pl.ds(my_x * m, m), :], xv_loc, in_sems.at[1])
        cp_g = pltpu.make_async_copy(g_ref, gv, in_sems.at[2])
        cp_send.start()
        cp_loc.start()
        cp_g.start()

        def half_off(off, role):
            base = (1 - s) * q if role == "extra" else s * q
            return base + off

        barrier = pltpu.get_barrier_semaphore()
        pl.semaphore_signal(
            barrier, inc=1, device_id=x_part,
            device_id_type=pl.DeviceIdType.MESH,
        )
        for dev in (y_part, z_part):
            pl.semaphore_signal(
                yz_sem, inc=1, device_id=dev,
                device_id_type=pl.DeviceIdType.MESH,
            )
        pl.semaphore_wait(barrier, 1)

        cp_send.wait()
        p1 = []
        coff = 0
        for c, (off, n, role) in enumerate(_SCHED):
            ho = half_off(off, role)
            r = pltpu.make_async_remote_copy(
                src_ref=xv_send.at[pl.ds(ho, n), :],
                dst_ref=comm_ref.at[pl.ds(coff, n), :],
                send_sem=p1_send.at[c],
                recv_sem=p1_recv.at[c],
                device_id=x_part,
                device_id_type=pl.DeviceIdType.MESH,
            )
            r.start()
            p1.append((r, coff))
            coff += n

        cp_loc.wait()
        cp_g.wait()
        pl.semaphore_wait(yz_sem, 2)

        p2 = []
        for c, (off, n, role) in enumerate(_SCHED):
            rdma, coff = p1[c]
            rdma.wait_recv()
            ho = half_off(off, role)
            red = xv_loc[pl.ds(ho, n), :] + comm_ref[pl.ds(coff, n), :]
            rinv = lax.rsqrt(jnp.mean(red * red, axis=-1, keepdims=True) + 1e-6)
            ov[pl.ds(ho, n), :] = red * rinv * gv[...]
            if role in ("y", "z"):
                fc = len(p2)
                r = pltpu.make_async_remote_copy(
                    src_ref=ov.at[pl.ds(ho, n), :],
                    dst_ref=ov.at[pl.ds(ho, n), :],
                    send_sem=p2_send.at[fc],
                    recv_sem=p2_recv.at[fc],
                    device_id=y_part if role == "y" else z_part,
                    device_id_type=pl.DeviceIdType.MESH,
                )
                r.start()
                p2.append(r)

        o1 = pltpu.make_async_copy(
            ov.at[pl.ds(s * q, q), :], out_ref.at[pl.ds(s * q, q), :],
            out_sems.at[0])
        o2a = pltpu.make_async_copy(
            ov.at[pl.ds((1 - s) * q + _FWD_HALF, TAU), :],
            out_ref.at[pl.ds((1 - s) * q + _FWD_HALF, TAU), :],
            out_sems.at[1])
        o2b = pltpu.make_async_copy(
            ov.at[pl.ds((1 - s) * q + q // 2 + _FWD_HALF, TAU), :],
            out_ref.at[pl.ds((1 - s) * q + q // 2 + _FWD_HALF, TAU), :],
            out_sems.at[4])
        o1.start()
        o2a.start()
        o2b.start()

        for r in p2[:-1]:
            r.wait_recv()
        o3 = pltpu.make_async_copy(
            ov.at[pl.ds((1 - s) * q, _FWD_HALF), :],
            out_ref.at[pl.ds((1 - s) * q, _FWD_HALF), :],
            out_sems.at[2])
        o3.start()
        p2[-1].wait_recv()
        o4 = pltpu.make_async_copy(
            ov.at[pl.ds((1 - s) * q + q // 2, _FWD_HALF), :],
            out_ref.at[pl.ds((1 - s) * q + q // 2, _FWD_HALF), :],
            out_sems.at[3])
        o4.start()

        o1.wait()
        o2a.wait()
        o2b.wait()
        o3.wait()
        o4.wait()
        for r, _ in p1:
            r.wait_send()
        for r in p2:
            r.wait_send()

    return pl.pallas_call(
        body,
        out_shape=jax.ShapeDtypeStruct((m, d), jnp.float32),
        in_specs=[
            pl.BlockSpec(memory_space=pl.ANY),
            pl.BlockSpec(memory_space=pl.ANY),
        ],
        out_specs=pl.BlockSpec(memory_space=pl.ANY),
        scratch_shapes=[
            pltpu.VMEM((m, d), jnp.float32),
            pltpu.VMEM((m, d), jnp.float32),
            pltpu.VMEM((d,), jnp.float32),
            pltpu.VMEM((m, d), jnp.float32),
            pltpu.VMEM((n_comm, d), jnp.float32),
            pltpu.SemaphoreType.DMA((3,)),
            pltpu.SemaphoreType.DMA((5,)),
            pltpu.SemaphoreType.REGULAR,
            pltpu.SemaphoreType.DMA((len(_SCHED),)),
            pltpu.SemaphoreType.DMA((len(_SCHED),)),
            pltpu.SemaphoreType.DMA((_N_FWD,)),
            pltpu.SemaphoreType.DMA((_N_FWD,)),
        ],
        compiler_params=pltpu.CompilerParams(collective_id=0),
    )(partial, gamma)
